# baseline (speedup 1.0000x reference)
import jax
import jax.numpy as jnp
from jax import lax
from jax.experimental import pallas as pl

_B = 4
_N = 20000
_NUM_CLASS = 3
_PRE = 1024
_POST = 512
_THRESH = 0.7
_R = 160
_NPAD = _R * 128
_NEG = float("-inf")


def _prep_kernel(cls_ref, box_ref, out_ref):
    c0 = cls_ref[0, 0]
    c1 = cls_ref[0, 1]
    c2 = cls_ref[0, 2]
    score = jnp.maximum(jnp.maximum(c0, c1), c2)
    label = jnp.where((c0 >= c1) & (c0 >= c2), 0.0,
                      jnp.where(c1 >= c2, 1.0, 2.0)).astype(jnp.float32)

    xc = box_ref[0, 0]
    yc = box_ref[0, 1]
    dx = box_ref[0, 3]
    dy = box_ref[0, 4]
    yaw = box_ref[0, 6]
    co = jnp.abs(jnp.cos(yaw))
    si = jnp.abs(jnp.sin(yaw))
    hw = 0.5 * (dx * co + dy * si)
    hh = 0.5 * (dx * si + dy * co)
    x1 = xc - hw
    y1 = yc - hh
    x2 = xc + hw
    y2 = yc + hh
    area = (x2 - x1) * (y2 - y1)

    out_ref[0, 0] = score
    out_ref[0, 1] = x1
    out_ref[0, 2] = y1
    out_ref[0, 3] = x2
    out_ref[0, 4] = y2
    out_ref[0, 5] = area
    out_ref[0, 6] = label
    out_ref[0, 7] = score


def _topk_kernel(planes_ref, flat_ref, out_ref):
    i0 = lax.broadcasted_iota(jnp.int32, (_R, 128), 0)
    i1 = lax.broadcasted_iota(jnp.int32, (_R, 128), 1)
    iotap = i0 * 128 + i1

    def body(j, s):
        m = jnp.max(s)
        pos = jnp.min(jnp.where(s == m, iotap, jnp.int32(_NPAD)))
        out_ref[0, pl.ds(j, 1), :] = flat_ref[0, pl.ds(pos, 1), :]
        return jnp.where(iotap == pos, _NEG, s)

    lax.fori_loop(0, _PRE, body, planes_ref[0, 0])


def _nms_kernel(rows_ref, cp_ref, out_ref):
    out_ref[...] = jnp.zeros((1, _POST, 16), jnp.float32)
    s0 = lax.broadcasted_iota(jnp.int32, (8, 128), 0)
    s1 = lax.broadcasted_iota(jnp.int32, (8, 128), 1)
    slot = s0 * 128 + s1
    ch = lax.broadcasted_iota(jnp.int32, (1, 16), 1)

    x1p = cp_ref[0, 9]
    y1p = cp_ref[0, 10]
    x2p = cp_ref[0, 11]
    y2p = cp_ref[0, 12]
    areap = cp_ref[0, 13]

    def cond(carry):
        i, k, _ = carry
        return (i < _PRE) & (k < _POST)

    def body(carry):
        i, k, keep = carry
        keep_i = jnp.max(jnp.where(slot == i, keep, 0.0)) > 0.5
        row = rows_ref[0, pl.ds(i, 1), :]
        x1b = row[0, 9]
        y1b = row[0, 10]
        x2b = row[0, 11]
        y2b = row[0, 12]
        areab = row[0, 13]
        iw = jnp.maximum(jnp.minimum(x2p, x2b) - jnp.maximum(x1p, x1b), 0.0)
        ih = jnp.maximum(jnp.minimum(y2p, y2b) - jnp.maximum(y1p, y1b), 0.0)
        inter = iw * ih
        iou = inter / (areap + areab - inter + 1e-6)
        keep_new = jnp.where((iou > _THRESH) & (slot > i), 0.0, keep)
        keep = jnp.where(keep_i, keep_new, keep)

        orow = jnp.where(ch == 8, row[:, 14:15] + 1.0,
                         jnp.where(ch == 7, row[:, 15:16], row))
        orow = jnp.where(ch >= 9, 0.0, orow)
        prev = out_ref[0, pl.ds(k, 1), :]
        out_ref[0, pl.ds(k, 1), :] = jnp.where(keep_i, orow, prev)
        return i + 1, k + keep_i.astype(jnp.int32), keep

    lax.while_loop(cond, body,
                   (jnp.int32(0), jnp.int32(0), jnp.ones((8, 128), jnp.float32)))


@jax.jit
def kernel(batch_box_preds, batch_cls_preds):
    b, n, _ = batch_box_preds.shape
    pad = _NPAD - n
    cls_p = jnp.pad(batch_cls_preds, ((0, 0), (0, pad), (0, 0)),
                    constant_values=-1.0)
    box_p = jnp.pad(batch_box_preds, ((0, 0), (0, pad), (0, 0)))
    cls_t = cls_p.transpose(0, 2, 1).reshape(b, _NUM_CLASS, _R, 128)
    box_t = box_p.transpose(0, 2, 1).reshape(b, 7, _R, 128)

    planes = pl.pallas_call(
        _prep_kernel,
        grid=(b,),
        in_specs=[
            pl.BlockSpec((1, _NUM_CLASS, _R, 128), lambda i: (i, 0, 0, 0)),
            pl.BlockSpec((1, 7, _R, 128), lambda i: (i, 0, 0, 0)),
        ],
        out_specs=pl.BlockSpec((1, 8, _R, 128), lambda i: (i, 0, 0, 0)),
        out_shape=jax.ShapeDtypeStruct((b, 8, _R, 128), jnp.float32),
    )(cls_t, box_t)

    flat_planes = planes.reshape(b, 8, _NPAD).transpose(0, 2, 1)
    flat_box = jnp.pad(box_p, ((0, 0), (0, 0), (0, 1)))
    flat16 = jnp.concatenate([flat_box, flat_planes], axis=-1)

    rows = pl.pallas_call(
        _topk_kernel,
        grid=(b,),
        in_specs=[
            pl.BlockSpec((1, 8, _R, 128), lambda i: (i, 0, 0, 0)),
            pl.BlockSpec((1, _NPAD, 16), lambda i: (i, 0, 0)),
        ],
        out_specs=pl.BlockSpec((1, _PRE, 16), lambda i: (i, 0, 0)),
        out_shape=jax.ShapeDtypeStruct((b, _PRE, 16), jnp.float32),
    )(planes, flat16)

    cplanes = rows.transpose(0, 2, 1).reshape(b, 16, 8, 128)

    out = pl.pallas_call(
        _nms_kernel,
        grid=(b,),
        in_specs=[
            pl.BlockSpec((1, _PRE, 16), lambda i: (i, 0, 0)),
            pl.BlockSpec((1, 16, 8, 128), lambda i: (i, 0, 0, 0)),
        ],
        out_specs=pl.BlockSpec((1, _POST, 16), lambda i: (i, 0, 0)),
        out_shape=jax.ShapeDtypeStruct((b, _POST, 16), jnp.float32),
    )(rows, cplanes)

    rois = out[:, :, 0:7]
    roi_scores = out[:, :, 7]
    roi_labels = out[:, :, 8].astype(jnp.int32)
    return rois, roi_scores, roi_labels
